# baseline (device time: 206608 ns/iter reference)
import jax
import jax.numpy as jnp
from jax import lax
from jax.experimental import pallas as pl
from jax.experimental.pallas import tpu as pltpu

N_DEV = 4
SQ = 2048
SKV = 2048
DM = 1024
HC = 8
DH = 128
BQ = 256
GW = 128
WIN = 512
NQB = SQ // BQ
SCALE = 0.08838834764831843
F32 = jnp.float32
BF16 = jnp.bfloat16


def kernel(x, Wq, K_ext, V_ext, Wo):
    x2 = (x.reshape(SQ, DM) * SCALE).astype(BF16)
    wq16 = Wq.astype(BF16)
    wo16 = Wo.astype(BF16)

    def body(x_ref, wq_hbm, k_hbm, v_hbm, wo_hbm, out_ref,
             comm, kbuf, vbuf, k16, v16, ctxb,
             bias0b, biasgb, biasmb, biaslb,
             send_sems, recv_sems, kv_sems, w_sems):
        my = lax.axis_index("i")
        left = lax.rem(my + N_DEV - 1, N_DEV)
        right = lax.rem(my + 1, N_DEV)

        cwq = pltpu.make_async_copy(wq_hbm, comm.at[0, :DM, :], w_sems.at[0])
        cwo = pltpu.make_async_copy(wo_hbm, comm.at[0, DM:, :], w_sems.at[1])
        cwq.start()
        cwo.start()

        def kv_copies(h):
            j = lax.rem(my - h + N_DEV, N_DEV)
            cps = []
            for hd in range(HC):
                cps.append(pltpu.make_async_copy(
                    k_hbm.at[my, :, j * HC + hd, :], kbuf.at[hd],
                    kv_sems.at[0]))
                cps.append(pltpu.make_async_copy(
                    v_hbm.at[my, :, j * HC + hd, :], vbuf.at[hd],
                    kv_sems.at[1]))
            return cps

        def kv_wait_convert(cps):
            for cp in cps:
                cp.wait()
            k16[...] = kbuf[...].astype(BF16)
            v16[...] = vbuf[...].astype(BF16)

        kv_pend = kv_copies(0)
        for cp in kv_pend:
            cp.start()

        barrier = pltpu.get_barrier_semaphore()
        for nbr in (left, right):
            pl.semaphore_signal(barrier, inc=1, device_id=(nbr,),
                                device_id_type=pl.DeviceIdType.MESH)
        pl.semaphore_wait(barrier, 2)

        cwq.wait()
        cwo.wait()
        kv_wait_convert(kv_pend)

        r0 = lax.broadcasted_iota(jnp.int32, (BQ, SKV), 0)
        c0 = lax.broadcasted_iota(jnp.int32, (BQ, SKV), 1)
        m0 = (jnp.abs(r0 - c0) <= 128) | (c0 < 32) | (r0 < 32)
        bias0b[...] = jnp.where(m0, 0.0, -1e9).astype(F32)
        cg = lax.broadcasted_iota(jnp.int32, (BQ, GW), 1)
        biasgb[...] = jnp.where(cg < 32, 0.0, -1e9).astype(F32)
        rw = lax.broadcasted_iota(jnp.int32, (BQ, WIN), 0)
        cw = lax.broadcasted_iota(jnp.int32, (BQ, WIN), 1)
        biasmb[...] = jnp.where(jnp.abs(cw - 128 - rw) <= 128,
                                0.0, -1e9).astype(F32)
        biaslb[...] = jnp.where(jnp.abs(cw - 256 - rw) <= 128,
                                0.0, -1e9).astype(F32)

        def attend(qh, k_sl, v_sl, bias):
            s = lax.dot_general(qh, k_sl, (((1,), (1,)), ((), ())),
                                preferred_element_type=F32)
            w = jnp.exp(s + bias)
            denom = jnp.sum(w, axis=1, keepdims=True)
            ctx = lax.dot_general(w.astype(BF16), v_sl,
                                  (((1,), (0,)), ((), ())),
                                  preferred_element_type=F32)
            return ctx / denom

        def attend2(qh, kg, kw, vg, vw, bias_w):
            sg = lax.dot_general(qh, kg, (((1,), (1,)), ((), ())),
                                 preferred_element_type=F32)
            sw = lax.dot_general(qh, kw, (((1,), (1,)), ((), ())),
                                 preferred_element_type=F32)
            wg = jnp.exp(sg + biasgb[...])
            ww = jnp.exp(sw + bias_w)
            denom = (jnp.sum(wg, axis=1, keepdims=True)
                     + jnp.sum(ww, axis=1, keepdims=True))
            ctx = (lax.dot_general(wg.astype(BF16), vg,
                                   (((1,), (0,)), ((), ())),
                                   preferred_element_type=F32)
                   + lax.dot_general(ww.astype(BF16), vw,
                                     (((1,), (0,)), ((), ())),
                                     preferred_element_type=F32))
            return ctx / denom

        for h in range(N_DEV):
            slot = h % 2
            if h < N_DEV - 1:
                rdma = pltpu.make_async_remote_copy(
                    src_ref=comm.at[slot],
                    dst_ref=comm.at[1 - slot],
                    send_sem=send_sems.at[slot],
                    recv_sem=recv_sems.at[1 - slot],
                    device_id=(right,),
                    device_id_type=pl.DeviceIdType.MESH,
                )
                rdma.start()
                kv_pend = kv_copies(h + 1)
                for cp in kv_pend:
                    cp.start()

            def accum(qs):
                contrib = lax.dot_general(
                    ctxb[...], comm[slot, DM:, :], (((1,), (0,)), ((), ())),
                    preferred_element_type=F32)
                if h == 0:
                    out_ref[pl.ds(qs, BQ), :] = contrib
                else:
                    out_ref[pl.ds(qs, BQ), :] = (
                        out_ref[pl.ds(qs, BQ), :] + contrib)

            def windowed_block(qs, start, bias_w):
                q_blk = lax.dot_general(
                    x_ref[pl.ds(qs, BQ), :], comm[slot, :DM, :],
                    (((1,), (0,)), ((), ())),
                    preferred_element_type=F32).astype(BF16)
                for hd in range(HC):
                    ctxb[:, hd * DH:(hd + 1) * DH] = attend2(
                        q_blk[:, hd * DH:(hd + 1) * DH],
                        k16[hd, :GW, :], k16[hd, pl.ds(start, WIN), :],
                        v16[hd, :GW, :], v16[hd, pl.ds(start, WIN), :],
                        bias_w).astype(BF16)
                accum(qs)

            q0 = lax.dot_general(
                x_ref[:BQ, :], comm[slot, :DM, :], (((1,), (0,)), ((), ())),
                preferred_element_type=F32).astype(BF16)
            for hd in range(HC):
                ctxb[:, hd * DH:(hd + 1) * DH] = attend(
                    q0[:, hd * DH:(hd + 1) * DH], k16[hd], v16[hd],
                    bias0b[...]).astype(BF16)
            accum(0)

            def qb_body(qb, carry):
                qs = qb * BQ
                windowed_block(qs, qs - 128, biasmb[...])
                return carry

            lax.fori_loop(1, NQB - 1, qb_body, 0)

            windowed_block((NQB - 1) * BQ, SKV - WIN, biaslb[...])

            if h < N_DEV - 1:
                rdma.wait()
                kv_wait_convert(kv_pend)

    out2 = pl.pallas_call(
        body,
        out_shape=jax.ShapeDtypeStruct((SQ, DM), F32),
        in_specs=[
            pl.BlockSpec(memory_space=pltpu.VMEM),
            pl.BlockSpec(memory_space=pl.ANY),
            pl.BlockSpec(memory_space=pl.ANY),
            pl.BlockSpec(memory_space=pl.ANY),
            pl.BlockSpec(memory_space=pl.ANY),
        ],
        out_specs=pl.BlockSpec(memory_space=pltpu.VMEM),
        scratch_shapes=[
            pltpu.VMEM((2, 2 * DM, DM), BF16),
            pltpu.VMEM((HC, SKV, DH), F32),
            pltpu.VMEM((HC, SKV, DH), F32),
            pltpu.VMEM((HC, SKV, DH), BF16),
            pltpu.VMEM((HC, SKV, DH), BF16),
            pltpu.VMEM((BQ, DM), BF16),
            pltpu.VMEM((BQ, SKV), F32),
            pltpu.VMEM((BQ, GW), F32),
            pltpu.VMEM((BQ, WIN), F32),
            pltpu.VMEM((BQ, WIN), F32),
            pltpu.SemaphoreType.DMA((2,)),
            pltpu.SemaphoreType.DMA((2,)),
            pltpu.SemaphoreType.DMA((2,)),
            pltpu.SemaphoreType.DMA((2,)),
        ],
        compiler_params=pltpu.CompilerParams(
            collective_id=0,
            vmem_limit_bytes=100 * 1024 * 1024,
        ),
    )(x2, wq16, K_ext, V_ext, wo16)
    return out2.reshape(1, SQ, DM)


# device time: 205695 ns/iter; 1.0044x vs baseline; 1.0044x over previous
import jax
import jax.numpy as jnp
from jax import lax
from jax.experimental import pallas as pl
from jax.experimental.pallas import tpu as pltpu

N_DEV = 4
SQ = 2048
SKV = 2048
DM = 1024
HC = 8
DH = 128
BQ = 256
GW = 128
WIN = 512
NQB = SQ // BQ
SCALE = 0.08838834764831843
F32 = jnp.float32
BF16 = jnp.bfloat16


def kernel(x, Wq, K_ext, V_ext, Wo):
    x2 = (x.reshape(SQ, DM) * SCALE).astype(BF16)
    wq16 = Wq.astype(BF16)
    wo16 = Wo.astype(BF16)

    def body(x_ref, wq_hbm, k_hbm, v_hbm, wo_hbm, out_ref,
             comm, kbuf, vbuf, k16, v16, ctxb,
             bias0b, biasgb, biasmb, biaslb,
             send_sems, recv_sems, kv_sems, w_sems):
        my = lax.axis_index("i")
        left = lax.rem(my + N_DEV - 1, N_DEV)
        right = lax.rem(my + 1, N_DEV)

        cwq = pltpu.make_async_copy(wq_hbm, comm.at[0, :DM, :], w_sems.at[0])
        cwo = pltpu.make_async_copy(wo_hbm, comm.at[0, DM:, :], w_sems.at[1])
        cwq.start()
        cwo.start()

        def kv_copies(h):
            j = lax.rem(my - h + N_DEV, N_DEV)
            cps = []
            for hd in range(HC):
                cps.append(pltpu.make_async_copy(
                    k_hbm.at[my, :, j * HC + hd, :], kbuf.at[hd],
                    kv_sems.at[0]))
                cps.append(pltpu.make_async_copy(
                    v_hbm.at[my, :, j * HC + hd, :], vbuf.at[hd],
                    kv_sems.at[1]))
            return cps

        def kv_wait_convert(cps):
            for cp in cps:
                cp.wait()
            k16[...] = kbuf[...].astype(BF16)
            v16[...] = vbuf[...].astype(BF16)

        kv_pend = kv_copies(0)
        for cp in kv_pend:
            cp.start()

        barrier = pltpu.get_barrier_semaphore()
        for nbr in (left, right):
            pl.semaphore_signal(barrier, inc=1, device_id=(nbr,),
                                device_id_type=pl.DeviceIdType.MESH)
        pl.semaphore_wait(barrier, 2)

        cwq.wait()
        cwo.wait()
        kv_wait_convert(kv_pend)

        r0 = lax.broadcasted_iota(jnp.int32, (BQ - 32, WIN), 0) + 32
        c0 = lax.broadcasted_iota(jnp.int32, (BQ - 32, WIN), 1)
        m0 = (c0 < 32) | (jnp.abs(r0 - c0) <= 128)
        bias0b[...] = jnp.where(m0, 0.0, -1e9).astype(F32)
        cg = lax.broadcasted_iota(jnp.int32, (BQ, GW), 1)
        biasgb[...] = jnp.where(cg < 32, 0.0, -1e9).astype(F32)
        rw = lax.broadcasted_iota(jnp.int32, (BQ, WIN), 0)
        cw = lax.broadcasted_iota(jnp.int32, (BQ, WIN), 1)
        biasmb[...] = jnp.where(jnp.abs(cw - 128 - rw) <= 128,
                                0.0, -1e9).astype(F32)
        biaslb[...] = jnp.where(jnp.abs(cw - 256 - rw) <= 128,
                                0.0, -1e9).astype(F32)

        def attend(qh, k_sl, v_sl, bias):
            s = lax.dot_general(qh, k_sl, (((1,), (1,)), ((), ())),
                                preferred_element_type=F32)
            w = jnp.exp(s + bias)
            denom = jnp.sum(w, axis=1, keepdims=True)
            ctx = lax.dot_general(w.astype(BF16), v_sl,
                                  (((1,), (0,)), ((), ())),
                                  preferred_element_type=F32)
            return ctx / denom

        def attend2(qh, kg, kw, vg, vw, bias_w):
            sg = lax.dot_general(qh, kg, (((1,), (1,)), ((), ())),
                                 preferred_element_type=F32)
            sw = lax.dot_general(qh, kw, (((1,), (1,)), ((), ())),
                                 preferred_element_type=F32)
            wg = jnp.exp(sg + biasgb[...])
            ww = jnp.exp(sw + bias_w)
            denom = (jnp.sum(wg, axis=1, keepdims=True)
                     + jnp.sum(ww, axis=1, keepdims=True))
            ctx = (lax.dot_general(wg.astype(BF16), vg,
                                   (((1,), (0,)), ((), ())),
                                   preferred_element_type=F32)
                   + lax.dot_general(ww.astype(BF16), vw,
                                     (((1,), (0,)), ((), ())),
                                     preferred_element_type=F32))
            return ctx / denom

        for h in range(N_DEV):
            slot = h % 2
            if h < N_DEV - 1:
                rdma = pltpu.make_async_remote_copy(
                    src_ref=comm.at[slot],
                    dst_ref=comm.at[1 - slot],
                    send_sem=send_sems.at[slot],
                    recv_sem=recv_sems.at[1 - slot],
                    device_id=(right,),
                    device_id_type=pl.DeviceIdType.MESH,
                )
                rdma.start()
                kv_pend = kv_copies(h + 1)
                for cp in kv_pend:
                    cp.start()

            def accum(qs):
                contrib = lax.dot_general(
                    ctxb[...], comm[slot, DM:, :], (((1,), (0,)), ((), ())),
                    preferred_element_type=F32)
                if h == 0:
                    out_ref[pl.ds(qs, BQ), :] = contrib
                else:
                    out_ref[pl.ds(qs, BQ), :] = (
                        out_ref[pl.ds(qs, BQ), :] + contrib)

            def windowed_block(qs, start, bias_w):
                q_blk = lax.dot_general(
                    x_ref[pl.ds(qs, BQ), :], comm[slot, :DM, :],
                    (((1,), (0,)), ((), ())),
                    preferred_element_type=F32).astype(BF16)
                for hd in range(HC):
                    ctxb[:, hd * DH:(hd + 1) * DH] = attend2(
                        q_blk[:, hd * DH:(hd + 1) * DH],
                        k16[hd, :GW, :], k16[hd, pl.ds(start, WIN), :],
                        v16[hd, :GW, :], v16[hd, pl.ds(start, WIN), :],
                        bias_w).astype(BF16)
                accum(qs)

            q0 = lax.dot_general(
                x_ref[:BQ, :], comm[slot, :DM, :], (((1,), (0,)), ((), ())),
                preferred_element_type=F32).astype(BF16)
            for hd in range(HC):
                sl = slice(hd * DH, (hd + 1) * DH)
                ctxb[:32, sl] = attend(
                    q0[:32, sl], k16[hd], v16[hd], 0.0).astype(BF16)
                ctxb[32:, sl] = attend(
                    q0[32:, sl], k16[hd, :WIN, :], v16[hd, :WIN, :],
                    bias0b[...]).astype(BF16)
            accum(0)

            def qb_body(qb, carry):
                qs = qb * BQ
                windowed_block(qs, qs - 128, biasmb[...])
                return carry

            lax.fori_loop(1, NQB - 1, qb_body, 0)

            windowed_block((NQB - 1) * BQ, SKV - WIN, biaslb[...])

            if h < N_DEV - 1:
                rdma.wait()
                kv_wait_convert(kv_pend)

    out2 = pl.pallas_call(
        body,
        out_shape=jax.ShapeDtypeStruct((SQ, DM), F32),
        in_specs=[
            pl.BlockSpec(memory_space=pltpu.VMEM),
            pl.BlockSpec(memory_space=pl.ANY),
            pl.BlockSpec(memory_space=pl.ANY),
            pl.BlockSpec(memory_space=pl.ANY),
            pl.BlockSpec(memory_space=pl.ANY),
        ],
        out_specs=pl.BlockSpec(memory_space=pltpu.VMEM),
        scratch_shapes=[
            pltpu.VMEM((2, 2 * DM, DM), BF16),
            pltpu.VMEM((HC, SKV, DH), F32),
            pltpu.VMEM((HC, SKV, DH), F32),
            pltpu.VMEM((HC, SKV, DH), BF16),
            pltpu.VMEM((HC, SKV, DH), BF16),
            pltpu.VMEM((BQ, DM), BF16),
            pltpu.VMEM((BQ - 32, WIN), F32),
            pltpu.VMEM((BQ, GW), F32),
            pltpu.VMEM((BQ, WIN), F32),
            pltpu.VMEM((BQ, WIN), F32),
            pltpu.SemaphoreType.DMA((2,)),
            pltpu.SemaphoreType.DMA((2,)),
            pltpu.SemaphoreType.DMA((2,)),
            pltpu.SemaphoreType.DMA((2,)),
        ],
        compiler_params=pltpu.CompilerParams(
            collective_id=0,
            vmem_limit_bytes=100 * 1024 * 1024,
        ),
    )(x2, wq16, K_ext, V_ext, wo16)
    return out2.reshape(1, SQ, DM)
